# baseline (speedup 1.0000x reference)
import jax
import jax.numpy as jnp
from jax.experimental import pallas as pl
from jax.experimental.pallas import tpu as pltpu

_ROWS = 512
_W = 1024


def _len_body(mask_ref, v_ref, maxlen_ref, mo_ref, vo_ref):
    m = mask_ref[...]
    maxlen = jnp.maximum(jnp.max(jnp.sum(m, axis=-1)), 1)
    maxlen_ref[0] = maxlen
    L = m.shape[-1]
    keep = jax.lax.broadcasted_iota(jnp.int32, (1, L), 1) < maxlen
    mo_ref[...] = jnp.logical_and(keep, m != 0)
    vo_ref[...] = jnp.where(keep, v_ref[...], 0.0)


def _x_body(maxlen_sref, x_ref, xo_ref):
    j = pl.program_id(1)
    maxlen = maxlen_sref[0]
    col = j * _W + jax.lax.broadcasted_iota(jnp.int32, (_ROWS, _W), 1)
    xo_ref[...] = jnp.where(col < maxlen, x_ref[...], 0.0)


def kernel(x, v, mask):
    B, C, L = x.shape
    Cv = v.shape[1]
    x2 = x.reshape(B * C, L)
    v2 = v.reshape(B * Cv, L)
    m2 = mask.reshape(B, L)

    maxlen, m_out2, v_out2 = pl.pallas_call(
        _len_body,
        in_specs=[
            pl.BlockSpec((B, L), lambda: (0, 0)),
            pl.BlockSpec((B * Cv, L), lambda: (0, 0)),
        ],
        out_specs=[
            pl.BlockSpec(memory_space=pltpu.SMEM),
            pl.BlockSpec((B, L), lambda: (0, 0)),
            pl.BlockSpec((B * Cv, L), lambda: (0, 0)),
        ],
        out_shape=[
            jax.ShapeDtypeStruct((1,), jnp.int32),
            jax.ShapeDtypeStruct((B, L), jnp.bool_),
            jax.ShapeDtypeStruct((B * Cv, L), v.dtype),
        ],
    )(m2, v2)

    grid_spec = pltpu.PrefetchScalarGridSpec(
        num_scalar_prefetch=1,
        grid=(B * C // _ROWS, L // _W),
        in_specs=[pl.BlockSpec((_ROWS, _W), lambda i, j, m: (i, j))],
        out_specs=pl.BlockSpec((_ROWS, _W), lambda i, j, m: (i, j)),
    )
    x_out2 = pl.pallas_call(
        _x_body,
        grid_spec=grid_spec,
        out_shape=jax.ShapeDtypeStruct((B * C, L), x.dtype),
    )(maxlen, x2)

    return (
        x_out2.reshape(B, C, L),
        v_out2.reshape(B, Cv, L),
        m_out2.reshape(B, 1, L),
    )
